# baseline (speedup 1.0000x reference)
import functools

import numpy as np
import jax
import jax.numpy as jnp
from jax import lax
from jax.experimental import pallas as pl
from jax.experimental.pallas import tpu as pltpu
from jax.experimental.pallas import tpu_sc as plsc

_LANES = 16


def _interleave_perm(embed):
    perm = np.empty(embed, dtype=np.int32)
    for j in range(embed):
        s = j // 32
        perm[j] = 32 * s + (j % 32) // 2 + 16 * (j % 2)
    return perm


@functools.lru_cache(maxsize=None)
def _build(B, MAXLEN, EMBED):
    info = plsc.get_sparse_core_info()
    NC, NS = info.num_cores, info.num_subcores
    NW = NC * NS
    assert B % NW == 0
    BPW = B // NW
    assert BPW % MAXLEN == 0
    CH = 400
    assert BPW % CH == 0 and CH % MAXLEN == 0
    NCHUNK = BPW // CH
    POSREP = CH // MAXLEN
    assert EMBED % 32 == 0
    SUBS = []
    off = 0
    while off < CH:
        sz = min(128, CH - off)
        SUBS.append((off, sz))
        off += sz

    mesh = plsc.VectorSubcoreMesh(core_axis_name="c", subcore_axis_name="s")

    @functools.partial(
        pl.kernel,
        mesh=mesh,
        compiler_params=pltpu.CompilerParams(
            use_tc_tiling_on_sc=False, needs_layout_passes=False
        ),
        out_type=jax.ShapeDtypeStruct((B, EMBED), jnp.float32),
        scratch_types=[
            pltpu.VMEM((BPW,), jnp.int32),
            pltpu.VMEM((CH, EMBED), jnp.float32),
            pltpu.VMEM((2, CH, EMBED), jnp.bfloat16),
            pltpu.VMEM((2, CH, EMBED), jnp.float32),
            pltpu.SemaphoreType.DMA,
            pltpu.SemaphoreType.DMA,
            pltpu.SemaphoreType.DMA,
            pltpu.SemaphoreType.DMA,
        ],
    )
    def embed(x_hbm, tok_hbm, pos_hbm, out_hbm, idx_v, pos2, bufs, obufs,
              g0, g1, s0, s1):
        gsem = (g0, g1)
        ssem = (s0, s1)
        wid = lax.axis_index("s") * NC + lax.axis_index("c")
        base = wid * BPW
        pltpu.sync_copy(x_hbm.at[pl.ds(base, BPW)], idx_v)
        for rep in range(POSREP):
            pltpu.sync_copy(pos_hbm, pos2.at[pl.ds(rep * MAXLEN, MAXLEN)])

        def issue_gathers(g):
            slot = g % 2
            co = g * CH
            return [
                pltpu.async_copy(
                    tok_hbm.at[idx_v.at[pl.ds(co + off, sz)]],
                    bufs.at[slot].at[pl.ds(off, sz)],
                    gsem[slot],
                )
                for off, sz in SUBS
            ]

        gathers = {0: issue_gathers(0)}
        stores = {}
        for g in range(NCHUNK):
            slot = g % 2
            for c in gathers.pop(g):
                c.wait()
            if g + 1 < NCHUNK:
                if g - 1 in stores:
                    stores.pop(g - 1).wait()
                gathers[g + 1] = issue_gathers(g + 1)

            buf = bufs.at[slot]
            obuf = obufs.at[slot]

            @plsc.parallel_loop(0, CH, 1, unroll=4)
            def add_row(r):
                for h in range(EMBED // 32):
                    u = plsc.bitcast(buf[r, pl.ds(32 * h, 32)], jnp.int32)
                    lo = plsc.bitcast(lax.shift_left(u, 16), jnp.float32)
                    hi = plsc.bitcast(
                        lax.bitwise_and(u, jnp.int32(-65536)), jnp.float32
                    )
                    sl_lo = pl.ds(32 * h, _LANES)
                    sl_hi = pl.ds(32 * h + _LANES, _LANES)
                    obuf[r, sl_lo] = lo + pos2[r, sl_lo]
                    obuf[r, sl_hi] = hi + pos2[r, sl_hi]

            stores[g] = pltpu.async_copy(
                obuf, out_hbm.at[pl.ds(base + g * CH, CH)], ssem[slot]
            )
        for g in sorted(stores):
            stores.pop(g).wait()

    return embed


def kernel(x, token_table, pos_table):
    batch, maxlen = x.shape
    embed_dim = token_table.shape[1]
    xf = x.reshape(-1).astype(jnp.int32)
    perm = _interleave_perm(embed_dim)
    tt = token_table[:, perm].astype(jnp.bfloat16)
    fn = _build(batch * maxlen, maxlen, embed_dim)
    out = fn(xf, tt, pos_table)
    return out.reshape(batch, maxlen, embed_dim)


# speedup vs baseline: 1.5496x; 1.5496x over previous
import functools

import numpy as np
import jax
import jax.numpy as jnp
from jax import lax
from jax.experimental import pallas as pl
from jax.experimental.pallas import tpu as pltpu
from jax.experimental.pallas import tpu_sc as plsc

_LANES = 16


def _deinterleave_perm(embed):
    perm = np.empty(embed, dtype=np.int32)
    for h in range(embed // 32):
        for l in range(16):
            perm[32 * h + l] = 32 * h + 2 * l
            perm[32 * h + 16 + l] = 32 * h + 2 * l + 1
    return perm


@functools.lru_cache(maxsize=None)
def _build(B, MAXLEN, EMBED):
    info = plsc.get_sparse_core_info()
    NC, NS = info.num_cores, info.num_subcores
    NW = NC * NS
    assert B % NW == 0
    BPW = B // NW
    assert BPW % MAXLEN == 0
    CH = 400
    assert BPW % CH == 0 and CH % MAXLEN == 0
    NCHUNK = BPW // CH
    POSREP = CH // MAXLEN
    assert EMBED % 32 == 0
    SUBS = []
    off = 0
    while off < CH:
        sz = min(128, CH - off)
        SUBS.append((off, sz))
        off += sz

    mesh = plsc.VectorSubcoreMesh(core_axis_name="c", subcore_axis_name="s")

    @functools.partial(
        pl.kernel,
        mesh=mesh,
        compiler_params=pltpu.CompilerParams(
            use_tc_tiling_on_sc=False, needs_layout_passes=False
        ),
        out_type=jax.ShapeDtypeStruct((B * EMBED,), jnp.float32),
        scratch_types=[
            pltpu.VMEM((BPW,), jnp.int32),
            pltpu.VMEM((CH, EMBED), jnp.float32),
            pltpu.VMEM((2, CH, EMBED), jnp.bfloat16),
            pltpu.VMEM((2, CH * EMBED), jnp.float32),
            pltpu.SemaphoreType.DMA,
            pltpu.SemaphoreType.DMA,
            pltpu.SemaphoreType.DMA,
            pltpu.SemaphoreType.DMA,
        ],
    )
    def embed(x_hbm, tok_hbm, pos_hbm, out_hbm, idx_v, pos2, bufs, obufs,
              g0, g1, s0, s1):
        gsem = (g0, g1)
        ssem = (s0, s1)
        wid = lax.axis_index("s") * NC + lax.axis_index("c")
        base = wid * BPW
        pltpu.sync_copy(x_hbm.at[pl.ds(base, BPW)], idx_v)
        for rep in range(POSREP):
            pltpu.sync_copy(pos_hbm, pos2.at[pl.ds(rep * MAXLEN, MAXLEN)])

        def issue_gathers(g):
            slot = g % 2
            co = g * CH
            return [
                pltpu.async_copy(
                    tok_hbm.at[idx_v.at[pl.ds(co + off, sz)]],
                    bufs.at[slot].at[pl.ds(off, sz)],
                    gsem[slot],
                )
                for off, sz in SUBS
            ]

        iota = lax.iota(jnp.int32, _LANES)
        gathers = {0: issue_gathers(0)}
        stores = {}
        for g in range(NCHUNK):
            slot = g % 2
            for c in gathers.pop(g):
                c.wait()
            if g + 1 < NCHUNK:
                if g - 1 in stores:
                    stores.pop(g - 1).wait()
                gathers[g + 1] = issue_gathers(g + 1)

            buf = bufs.at[slot]
            obuf = obufs.at[slot]

            @plsc.parallel_loop(0, CH, 1, unroll=4)
            def add_row(r):
                for h in range(EMBED // 32):
                    u = plsc.bitcast(buf[r, pl.ds(32 * h, 32)], jnp.int32)
                    even = plsc.bitcast(lax.shift_left(u, 16), jnp.float32)
                    odd = plsc.bitcast(
                        lax.bitwise_and(u, jnp.int32(-65536)), jnp.float32
                    )
                    se = even + pos2[r, pl.ds(32 * h, _LANES)]
                    so = odd + pos2[r, pl.ds(32 * h + _LANES, _LANES)]
                    ob = r * EMBED + 32 * h + 2 * iota
                    plsc.store_scatter(obuf, [ob], se)
                    plsc.store_scatter(obuf, [ob + 1], so)

            stores[g] = pltpu.async_copy(
                obuf,
                out_hbm.at[pl.ds((base + g * CH) * EMBED, CH * EMBED)],
                ssem[slot],
            )
        for g in sorted(stores):
            stores.pop(g).wait()

    return embed


def kernel(x, token_table, pos_table):
    batch, maxlen = x.shape
    embed_dim = token_table.shape[1]
    xf = x.reshape(-1).astype(jnp.int32)
    tt = token_table.astype(jnp.bfloat16)
    perm = _deinterleave_perm(embed_dim)
    pos_p = pos_table[:, perm]
    fn = _build(batch * maxlen, maxlen, embed_dim)
    out = fn(xf, tt, pos_p)
    return out.reshape(batch, maxlen, embed_dim)


# speedup vs baseline: 1.8738x; 1.2092x over previous
import functools

import jax
import jax.numpy as jnp
from jax import lax
from jax.experimental import pallas as pl
from jax.experimental.pallas import tpu as pltpu
from jax.experimental.pallas import tpu_sc as plsc

_LANES = 16


@functools.lru_cache(maxsize=None)
def _build(B, MAXLEN, EMBED):
    info = plsc.get_sparse_core_info()
    NC, NS = info.num_cores, info.num_subcores
    NW = NC * NS
    assert B % NW == 0
    BPW = B // NW
    assert BPW % MAXLEN == 0
    CH = 128
    assert BPW % CH == 0 and (CH // 2) % 8 == 0
    NCHUNK = BPW // CH
    EV = EMBED // _LANES
    PAIR = 2 * EMBED
    SUBS = []
    off = 0
    while off < CH:
        sz = min(128, CH - off)
        SUBS.append((off, sz))
        off += sz

    mesh = plsc.VectorSubcoreMesh(core_axis_name="c", subcore_axis_name="s")

    @functools.partial(
        pl.kernel,
        mesh=mesh,
        compiler_params=pltpu.CompilerParams(
            use_tc_tiling_on_sc=True, needs_layout_passes=False
        ),
        out_type=jax.ShapeDtypeStruct((B // 2, PAIR), jnp.float32),
        scratch_types=[
            pltpu.VMEM((BPW + _LANES,), jnp.int32),
            pltpu.VMEM((BPW,), jnp.int32),
            pltpu.VMEM((MAXLEN, EMBED), jnp.float32),
            pltpu.VMEM((2, CH, PAIR), jnp.float32),
            pltpu.VMEM((2, CH // 2, PAIR), jnp.float32),
            pltpu.SemaphoreType.DMA,
            pltpu.SemaphoreType.DMA,
            pltpu.SemaphoreType.DMA,
            pltpu.SemaphoreType.DMA,
        ],
    )
    def embed(x_hbm, tok_hbm, pos_hbm, out_hbm, idx_v, idxp_v, pos2, bufs,
              obufs, g0, g1, s0, s1):
        gsem = (g0, g1)
        ssem = (s0, s1)
        wid = lax.axis_index("s") * NC + lax.axis_index("c")
        base = pl.multiple_of(wid * BPW, BPW)
        pltpu.sync_copy(x_hbm.at[pl.ds(base, BPW)], idx_v.at[pl.ds(0, BPW)])
        pltpu.sync_copy(pos_hbm, pos2)

        @plsc.parallel_loop(0, BPW // _LANES, 1, unroll=8)
        def make_idxp(i):
            sl = pl.ds(i * _LANES, _LANES)
            idxp_v[sl] = lax.shift_right_logical(idx_v[sl], 1)

        def issue_gathers(g):
            slot = g % 2
            co = g * CH
            return [
                pltpu.async_copy(
                    tok_hbm.at[idxp_v.at[pl.ds(co + off, sz)]],
                    bufs.at[slot].at[pl.ds(off, sz)],
                    gsem[slot],
                )
                for off, sz in SUBS
            ]

        gathers = {0: issue_gathers(0)}
        stores = {}
        for g in range(NCHUNK):
            slot = g % 2
            for c in gathers.pop(g):
                c.wait()
            if g + 1 < NCHUNK:
                if g - 1 in stores:
                    stores.pop(g - 1).wait()
                gathers[g + 1] = issue_gathers(g + 1)

            buf = bufs.at[slot]
            obuf = obufs.at[slot]
            co = g * CH

            @plsc.parallel_loop(0, CH, 1, unroll=4)
            def add_row(r):
                v = idx_v[pl.ds(co + r, _LANES)][0]
                hoff = (v & 1) * EMBED
                orow = lax.shift_right_logical(r, 1)
                ooff = (r & 1) * EMBED
                p = lax.rem(co + r, MAXLEN)
                for k in range(EV):
                    sl = pl.ds(k * _LANES, _LANES)
                    obuf[orow, pl.ds(ooff + k * _LANES, _LANES)] = (
                        buf[r, pl.ds(hoff + k * _LANES, _LANES)] + pos2[p, sl]
                    )

            stores[g] = pltpu.async_copy(
                obuf,
                out_hbm.at[
                    pl.ds(pl.multiple_of((base + co) // 2, CH // 2), CH // 2)
                ],
                ssem[slot],
            )
        for g in sorted(stores):
            stores.pop(g).wait()

    return embed


def kernel(x, token_table, pos_table):
    batch, maxlen = x.shape
    vocab, embed_dim = token_table.shape
    xf = x.reshape(-1).astype(jnp.int32)
    tbl2 = token_table.reshape(vocab // 2, 2 * embed_dim)
    fn = _build(batch * maxlen, maxlen, embed_dim)
    out = fn(xf, tbl2, pos_table)
    return out.reshape(batch, maxlen, embed_dim)


# speedup vs baseline: 2.2378x; 1.1943x over previous
import functools

import jax
import jax.numpy as jnp
from jax import lax
from jax.experimental import pallas as pl
from jax.experimental.pallas import tpu as pltpu
from jax.experimental.pallas import tpu_sc as plsc

_LANES = 16


@functools.lru_cache(maxsize=None)
def _build(B, MAXLEN, EMBED):
    info = plsc.get_sparse_core_info()
    NC, NS = info.num_cores, info.num_subcores
    NW = NC * NS
    assert B % NW == 0
    BPW = B // NW
    assert BPW % MAXLEN == 0
    CH = 128
    assert BPW % CH == 0 and CH % 8 == 0
    NCHUNK = BPW // CH
    EV = EMBED // _LANES
    PAD = 2 * EMBED

    mesh = plsc.VectorSubcoreMesh(core_axis_name="c", subcore_axis_name="s")

    @functools.partial(
        pl.kernel,
        mesh=mesh,
        compiler_params=pltpu.CompilerParams(
            use_tc_tiling_on_sc=True, needs_layout_passes=False
        ),
        out_type=jax.ShapeDtypeStruct((B, EMBED), jnp.float32),
        scratch_types=[
            pltpu.VMEM((BPW,), jnp.int32),
            pltpu.VMEM((MAXLEN, EMBED), jnp.float32),
            pltpu.VMEM((2, CH, PAD), jnp.float32),
            pltpu.VMEM((2, CH, EMBED), jnp.float32),
            pltpu.SemaphoreType.DMA,
            pltpu.SemaphoreType.DMA,
            pltpu.SemaphoreType.DMA,
            pltpu.SemaphoreType.DMA,
        ],
    )
    def embed(x_hbm, tok_hbm, pos_hbm, out_hbm, idx_v, pos2, bufs, obufs,
              g0, g1, s0, s1):
        gsem = (g0, g1)
        ssem = (s0, s1)
        wid = lax.axis_index("s") * NC + lax.axis_index("c")
        base = pl.multiple_of(wid * BPW, BPW)
        pltpu.sync_copy(x_hbm.at[pl.ds(base, BPW)], idx_v)
        pltpu.sync_copy(pos_hbm, pos2)

        def issue_gather(g):
            slot = g % 2
            return pltpu.async_copy(
                tok_hbm.at[idx_v.at[pl.ds(g * CH, CH)]],
                bufs.at[slot],
                gsem[slot],
            )

        gathers = {0: issue_gather(0)}
        stores = {}
        for g in range(NCHUNK):
            slot = g % 2
            gathers.pop(g).wait()
            if g + 1 < NCHUNK:
                if g - 1 in stores:
                    stores.pop(g - 1).wait()
                gathers[g + 1] = issue_gather(g + 1)

            buf = bufs.at[slot]
            obuf = obufs.at[slot]
            co = g * CH

            @plsc.parallel_loop(0, CH, 1, unroll=4)
            def add_row(r):
                p = lax.rem(co + r, MAXLEN)
                for k in range(EV):
                    sl = pl.ds(k * _LANES, _LANES)
                    obuf[r, sl] = buf[r, sl] + pos2[p, sl]

            stores[g] = pltpu.async_copy(
                obuf,
                out_hbm.at[pl.ds(pl.multiple_of(base + co, CH), CH)],
                ssem[slot],
            )
        for g in sorted(stores):
            stores.pop(g).wait()

    return embed


def kernel(x, token_table, pos_table):
    batch, maxlen = x.shape
    vocab, embed_dim = token_table.shape
    xf = x.reshape(-1).astype(jnp.int32)
    tbl_pad = jnp.pad(token_table, ((0, 0), (0, embed_dim)))
    fn = _build(batch * maxlen, maxlen, embed_dim)
    out = fn(xf, tbl_pad, pos_table)
    return out.reshape(batch, maxlen, embed_dim)


# speedup vs baseline: 2.2783x; 1.0181x over previous
import functools

import jax
import jax.numpy as jnp
from jax import lax
from jax.experimental import pallas as pl
from jax.experimental.pallas import tpu as pltpu
from jax.experimental.pallas import tpu_sc as plsc

_LANES = 16


@functools.lru_cache(maxsize=None)
def _build(B, MAXLEN, EMBED):
    info = plsc.get_sparse_core_info()
    NC, NS = info.num_cores, info.num_subcores
    NW = NC * NS
    assert B % NW == 0
    BPW = B // NW
    assert BPW % MAXLEN == 0
    CH = MAXLEN
    assert BPW % CH == 0 and CH % 8 == 0
    NCHUNK = BPW // CH
    EV = EMBED // _LANES
    PAD = 2 * EMBED

    mesh = plsc.VectorSubcoreMesh(core_axis_name="c", subcore_axis_name="s")

    @functools.partial(
        pl.kernel,
        mesh=mesh,
        compiler_params=pltpu.CompilerParams(
            use_tc_tiling_on_sc=True, needs_layout_passes=False
        ),
        out_type=jax.ShapeDtypeStruct((B, EMBED), jnp.float32),
        scratch_types=[
            pltpu.VMEM((BPW,), jnp.int32),
            pltpu.VMEM((MAXLEN, EMBED), jnp.float32),
            pltpu.VMEM((2, CH, PAD), jnp.float32),
            pltpu.VMEM((CH, EMBED), jnp.float32),
            pltpu.SemaphoreType.DMA,
            pltpu.SemaphoreType.DMA,
            pltpu.SemaphoreType.DMA,
            pltpu.SemaphoreType.DMA,
        ],
    )
    def embed(x_hbm, tok_hbm, pos_hbm, out_hbm, idx_v, pos2, bufs, obufs,
              g0, g1, s0, s1):
        gsem = (g0, g1)
        ssem = (s0, s1)
        wid = lax.axis_index("s") * NC + lax.axis_index("c")
        base = pl.multiple_of(wid * BPW, BPW)
        pltpu.sync_copy(x_hbm.at[pl.ds(base, BPW)], idx_v)
        pltpu.sync_copy(pos_hbm, pos2)

        def issue_gathers(g):
            slot = g % 2
            return [
                pltpu.async_copy(
                    tok_hbm.at[idx_v.at[pl.ds(g * CH + off, sz)]],
                    bufs.at[slot].at[pl.ds(off, sz)],
                    gsem[slot],
                )
                for off, sz in ((0, 128), (128, CH - 128))
            ]

        gathers = {0: issue_gathers(0)}
        stores = {}
        for g in range(NCHUNK):
            slot = g % 2
            for c in gathers.pop(g):
                c.wait()
            if g + 1 < NCHUNK:
                gathers[g + 1] = issue_gathers(g + 1)
            if g - 1 in stores:
                stores.pop(g - 1).wait()

            buf = bufs.at[slot]
            obuf = obufs
            co = g * CH

            @plsc.parallel_loop(0, CH, 1, unroll=4)
            def add_row(r):
                for k in range(EV):
                    sl = pl.ds(k * _LANES, _LANES)
                    obuf[r, sl] = buf[r, sl] + pos2[r, sl]

            stores[g] = pltpu.async_copy(
                obuf,
                out_hbm.at[pl.ds(pl.multiple_of(base + co, CH), CH)],
                ssem[slot],
            )
        for g in sorted(stores):
            stores.pop(g).wait()

    return embed


def kernel(x, token_table, pos_table):
    batch, maxlen = x.shape
    vocab, embed_dim = token_table.shape
    xf = x.reshape(-1).astype(jnp.int32)
    tbl_pad = jnp.pad(token_table, ((0, 0), (0, embed_dim)))
    fn = _build(batch * maxlen, maxlen, embed_dim)
    out = fn(xf, tbl_pad, pos_table)
    return out.reshape(batch, maxlen, embed_dim)


# speedup vs baseline: 2.2814x; 1.0014x over previous
import functools

import jax
import jax.numpy as jnp
from jax import lax
from jax.experimental import pallas as pl
from jax.experimental.pallas import tpu as pltpu
from jax.experimental.pallas import tpu_sc as plsc

_LANES = 16


@functools.lru_cache(maxsize=None)
def _build(B, MAXLEN, EMBED):
    info = plsc.get_sparse_core_info()
    NC, NS = info.num_cores, info.num_subcores
    NW = NC * NS
    assert B % NW == 0
    BPW = B // NW
    assert BPW % MAXLEN == 0
    CH = MAXLEN
    assert BPW % CH == 0 and CH % 8 == 0
    NCHUNK = BPW // CH
    EV = EMBED // _LANES
    PAD = 2 * EMBED

    mesh = plsc.VectorSubcoreMesh(core_axis_name="c", subcore_axis_name="s")

    @functools.partial(
        pl.kernel,
        mesh=mesh,
        compiler_params=pltpu.CompilerParams(
            use_tc_tiling_on_sc=True, needs_layout_passes=False
        ),
        out_type=jax.ShapeDtypeStruct((B, EMBED), jnp.float32),
        scratch_types=[
            pltpu.VMEM((BPW,), jnp.int32),
            pltpu.VMEM((MAXLEN, EMBED), jnp.float32),
            pltpu.VMEM((2, CH, PAD), jnp.float32),
            pltpu.VMEM((CH, EMBED), jnp.float32),
            pltpu.SemaphoreType.DMA,
            pltpu.SemaphoreType.DMA,
            pltpu.SemaphoreType.DMA,
            pltpu.SemaphoreType.DMA,
        ],
    )
    def embed(x_hbm, tok_hbm, pos_hbm, out_hbm, idx_v, pos2, bufs, obufs,
              g0, g1, s0, s1):
        gsem = (g0, g1)
        ssem = (s0, s1)
        wid = lax.axis_index("s") * NC + lax.axis_index("c")
        base = pl.multiple_of(wid * BPW, BPW)
        pltpu.sync_copy(x_hbm.at[pl.ds(base, BPW)], idx_v)
        pltpu.sync_copy(pos_hbm, pos2)

        def issue_gathers(g):
            slot = g % 2
            return [
                pltpu.async_copy(
                    tok_hbm.at[idx_v.at[pl.ds(g * CH + off, sz)]],
                    bufs.at[slot].at[pl.ds(off, sz)],
                    gsem[slot],
                )
                for off, sz in ((0, 128), (128, CH - 128))
            ]

        gathers = {0: issue_gathers(0)}
        stores = {}
        for g in range(NCHUNK):
            slot = g % 2
            if g + 1 < NCHUNK:
                gathers[g + 1] = issue_gathers(g + 1)
            for c in gathers.pop(g):
                c.wait()
            if g - 1 in stores:
                stores.pop(g - 1).wait()

            buf = bufs.at[slot]
            obuf = obufs
            co = g * CH

            @plsc.parallel_loop(0, CH, 1, unroll=8)
            def add_row(r):
                for k in range(EV):
                    sl = pl.ds(k * _LANES, _LANES)
                    obuf[r, sl] = buf[r, sl] + pos2[r, sl]

            stores[g] = pltpu.async_copy(
                obuf,
                out_hbm.at[pl.ds(pl.multiple_of(base + co, CH), CH)],
                ssem[slot],
            )
        for g in sorted(stores):
            stores.pop(g).wait()

    return embed


def kernel(x, token_table, pos_table):
    batch, maxlen = x.shape
    vocab, embed_dim = token_table.shape
    xf = x.reshape(-1).astype(jnp.int32)
    tbl_pad = jnp.pad(token_table, ((0, 0), (0, embed_dim)))
    fn = _build(batch * maxlen, maxlen, embed_dim)
    out = fn(xf, tbl_pad, pos_table)
    return out.reshape(batch, maxlen, embed_dim)
